# Initial kernel scaffold; baseline (speedup 1.0000x reference)
import jax
import jax.numpy as jnp
from jax.experimental import pallas as pl


def kernel(item_emb, edge_index, edge_vals):
    raise NotImplementedError("write your pallas kernel here")


# speedup vs baseline: 3.8434x; 3.8434x over previous
import functools

import jax
import jax.numpy as jnp
from jax import lax
from jax.experimental import pallas as pl
from jax.experimental.pallas import tpu as pltpu
from jax.experimental.pallas import tpu_sc as plsc

N = 10000
D = 128
E = 320000
H = D // 2
NT = 16
CH = 128
EPT = -(-E // NT)
NCH = -(-EPT // CH)
EPT_PAD = NCH * CH
NP = 10240
RPT = NP // NT


def _sc_spmm3(x0, colp, rowp, valp, zeros):
    mesh = plsc.VectorSubcoreMesh(core_axis_name="c", subcore_axis_name="s")
    out = jax.ShapeDtypeStruct((2 * NP, H), jnp.float32)

    @functools.partial(
        pl.kernel,
        out_type=(out, out, out),
        mesh=mesh,
        compiler_params=pltpu.CompilerParams(
            needs_layout_passes=False, use_tc_tiling_on_sc=False),
        scratch_types=[
            pltpu.VMEM((NCH, CH), jnp.int32),
            pltpu.VMEM((NCH, CH), jnp.int32),
            pltpu.VMEM((NCH * CH,), jnp.float32),
            pltpu.VMEM((CH, H), jnp.float32),
            pltpu.VMEM_SHARED((NP, H), jnp.float32),
            pltpu.SemaphoreType.DMA,
        ],
    )
    def k(x0_h, colp_h, rowp_h, valp_h, zeros_h, y1_h, y2_h, y3_h,
          col_v, row_v, val_v, rows_v, acc, sem):
        c = lax.axis_index("c")
        s = lax.axis_index("s")

        pltpu.sync_copy(colp_h.at[s], col_v)
        pltpu.sync_copy(rowp_h.at[s], row_v)
        pltpu.sync_copy(valp_h.at[s], val_v)

        off = jnp.full((16,), c * NP, jnp.int32)

        @pl.loop(0, NCH * CH // 16)
        def _offset(i):
            j = i // (CH // 16)
            k16 = (i % (CH // 16)) * 16
            col_v[j, pl.ds(k16, 16)] = col_v[j, pl.ds(k16, 16)] + off

        for src_h, dst_h in ((x0_h, y1_h), (y1_h, y2_h), (y2_h, y3_h)):
            pltpu.sync_copy(zeros_h, acc.at[pl.ds(s * RPT, RPT)])
            plsc.subcore_barrier()

            @pl.loop(0, NCH)
            def _chunk(j):
                pltpu.async_copy(src_h.at[col_v.at[j]], rows_v, sem).wait()
                vbase = jnp.full((16,), j * CH, jnp.int32)

                @pl.loop(0, CH)
                def _scale(e):
                    v = plsc.load_gather(val_v, [vbase + e])
                    for kk in range(H // 16):
                        rows_v[e, pl.ds(kk * 16, 16)] = (
                            rows_v[e, pl.ds(kk * 16, 16)] * v)

                pltpu.sync_copy(rows_v, acc.at[row_v.at[j]], add=True)

            plsc.subcore_barrier()
            pltpu.sync_copy(acc.at[pl.ds(s * RPT, RPT)],
                            dst_h.at[pl.ds(c * NP + s * RPT, RPT)])
            plsc.subcore_barrier()

    return k(x0, colp, rowp, valp, zeros)


def _assemble(item_emb, y1, y2, y3):
    BN = 1000

    def body(x0_r, y1_r, y2_r, y3_r, e1_r, e2_r, e3_r, tot_r):
        e1 = jnp.concatenate([y1_r[0], y1_r[1]], axis=-1)
        e2 = jnp.concatenate([y2_r[0], y2_r[1]], axis=-1)
        e3 = jnp.concatenate([y3_r[0], y3_r[1]], axis=-1)
        e1_r[...] = e1
        e2_r[...] = e2
        e3_r[...] = e3
        tot_r[...] = x0_r[...] + e1 + e2 + e3

    half = pl.BlockSpec((2, BN, H), lambda i: (0, i, 0))
    full = pl.BlockSpec((BN, D), lambda i: (i, 0))
    outs = jax.ShapeDtypeStruct((N, D), jnp.float32)
    return pl.pallas_call(
        body,
        grid=(N // BN,),
        in_specs=[full, half, half, half],
        out_specs=[full, full, full, full],
        out_shape=(outs, outs, outs, outs),
    )(item_emb,
      y1.reshape(2, NP, H), y2.reshape(2, NP, H), y3.reshape(2, NP, H))


def kernel(item_emb, edge_index, edge_vals):
    rpad = ((0, NP - N), (0, 0))
    x0 = jnp.concatenate([jnp.pad(item_emb[:, :H], rpad),
                          jnp.pad(item_emb[:, H:], rpad)], axis=0)

    pad = NT * EPT_PAD - E
    col = jnp.concatenate([edge_index[1], jnp.zeros((pad,), jnp.int32)])
    row = jnp.concatenate([edge_index[0], jnp.zeros((pad,), jnp.int32)])
    val = jnp.concatenate([edge_vals, jnp.zeros((pad,), jnp.float32)])
    colp = col.reshape(NT, NCH, CH)
    rowp = row.reshape(NT, NCH, CH)
    valp = val.reshape(NT, NCH * CH)
    zeros = jnp.zeros((RPT, H), jnp.float32)

    y1, y2, y3 = _sc_spmm3(x0, colp, rowp, valp, zeros)
    e1, e2, e3, total = _assemble(item_emb, y1, y2, y3)
    return (total, (item_emb, e1, e2, e3))


# speedup vs baseline: 7.1848x; 1.8694x over previous
import functools

import jax
import jax.numpy as jnp
from jax import lax
from jax.experimental import pallas as pl
from jax.experimental.pallas import tpu as pltpu
from jax.experimental.pallas import tpu_sc as plsc

N = 10000
D = 128
E = 320000
H = D // 2
NT = 16
CH = 128
EPT = -(-E // NT)
NCH = 2 * -(-EPT // (2 * CH))
EPT_PAD = NCH * CH
NP = 10240
RPT = NP // NT


def _sc_spmm3(x0, colp, rowp, valp, zeros):
    mesh = plsc.VectorSubcoreMesh(core_axis_name="c", subcore_axis_name="s")
    out = jax.ShapeDtypeStruct((2 * NP, H), jnp.float32)

    @functools.partial(
        pl.kernel,
        out_type=(out, out, out),
        mesh=mesh,
        compiler_params=pltpu.CompilerParams(
            needs_layout_passes=False, use_tc_tiling_on_sc=False),
        scratch_types=[
            pltpu.VMEM((NCH, CH), jnp.int32),
            pltpu.VMEM((NCH, CH), jnp.int32),
            pltpu.VMEM((NCH * CH,), jnp.float32),
            pltpu.VMEM((CH, H), jnp.float32),
            pltpu.VMEM((CH, H), jnp.float32),
            pltpu.VMEM_SHARED((NP, H), jnp.float32),
            pltpu.SemaphoreType.DMA,
            pltpu.SemaphoreType.DMA,
            pltpu.SemaphoreType.DMA,
        ],
    )
    def k(x0_h, colp_h, rowp_h, valp_h, zeros_h, y1_h, y2_h, y3_h,
          col_v, row_v, val_v, rows_a, rows_b, acc, gsem_a, gsem_b, ssem):
        c = lax.axis_index("c")
        s = lax.axis_index("s")

        pltpu.sync_copy(colp_h.at[s], col_v)
        pltpu.sync_copy(rowp_h.at[s], row_v)
        pltpu.sync_copy(valp_h.at[s], val_v)

        off = jnp.full((16,), c * NP, jnp.int32)

        @pl.loop(0, NCH * CH // 16)
        def _offset(i):
            j = i // (CH // 16)
            k16 = (i % (CH // 16)) * 16
            col_v[j, pl.ds(k16, 16)] = col_v[j, pl.ds(k16, 16)] + off

        def scale(buf, j):
            vbase = jnp.full((16,), j * CH, jnp.int32)

            @functools.partial(plsc.parallel_loop, 0, CH, unroll=4)
            def _scale(e):
                v = plsc.load_gather(val_v, [vbase + e])
                for kk in range(H // 16):
                    buf[e, pl.ds(kk * 16, 16)] = (
                        buf[e, pl.ds(kk * 16, 16)] * v)

        for src_h, dst_h in ((x0_h, y1_h), (y1_h, y2_h), (y2_h, y3_h)):
            pltpu.sync_copy(zeros_h, acc.at[pl.ds(s * RPT, RPT)])
            plsc.subcore_barrier()

            pltpu.async_copy(src_h.at[col_v.at[0]], rows_a, gsem_a)
            pltpu.async_copy(src_h.at[col_v.at[1]], rows_b, gsem_b)

            @pl.loop(0, NCH // 2)
            def _pair(i):
                for off, buf, gsem in ((0, rows_a, gsem_a),
                                       (1, rows_b, gsem_b)):
                    j = 2 * i + off
                    pltpu.make_async_copy(
                        src_h.at[col_v.at[j]], buf, gsem).wait()
                    scale(buf, j)
                    sc = pltpu.async_copy(
                        buf, acc.at[row_v.at[j]], ssem, add=True)
                    sc.wait()

                    @pl.when(j + 2 < NCH)
                    def _prefetch():
                        pltpu.async_copy(
                            src_h.at[col_v.at[j + 2]], buf, gsem)

            plsc.subcore_barrier()
            pltpu.sync_copy(acc.at[pl.ds(s * RPT, RPT)],
                            dst_h.at[pl.ds(c * NP + s * RPT, RPT)])
            plsc.subcore_barrier()

    return k(x0, colp, rowp, valp, zeros)


def _assemble(item_emb, y1, y2, y3):
    BN = 1000

    def body(x0_r, y1_r, y2_r, y3_r, e1_r, e2_r, e3_r, tot_r):
        e1 = jnp.concatenate([y1_r[0], y1_r[1]], axis=-1)
        e2 = jnp.concatenate([y2_r[0], y2_r[1]], axis=-1)
        e3 = jnp.concatenate([y3_r[0], y3_r[1]], axis=-1)
        e1_r[...] = e1
        e2_r[...] = e2
        e3_r[...] = e3
        tot_r[...] = x0_r[...] + e1 + e2 + e3

    half = pl.BlockSpec((2, BN, H), lambda i: (0, i, 0))
    full = pl.BlockSpec((BN, D), lambda i: (i, 0))
    outs = jax.ShapeDtypeStruct((N, D), jnp.float32)
    return pl.pallas_call(
        body,
        grid=(N // BN,),
        in_specs=[full, half, half, half],
        out_specs=[full, full, full, full],
        out_shape=(outs, outs, outs, outs),
    )(item_emb,
      y1.reshape(2, NP, H), y2.reshape(2, NP, H), y3.reshape(2, NP, H))


def kernel(item_emb, edge_index, edge_vals):
    rpad = ((0, NP - N), (0, 0))
    x0 = jnp.concatenate([jnp.pad(item_emb[:, :H], rpad),
                          jnp.pad(item_emb[:, H:], rpad)], axis=0)

    pad = NT * EPT_PAD - E
    col = jnp.concatenate([edge_index[1], jnp.zeros((pad,), jnp.int32)])
    row = jnp.concatenate([edge_index[0], jnp.zeros((pad,), jnp.int32)])
    val = jnp.concatenate([edge_vals, jnp.zeros((pad,), jnp.float32)])
    colp = col.reshape(NT, NCH, CH)
    rowp = row.reshape(NT, NCH, CH)
    valp = val.reshape(NT, NCH * CH)
    zeros = jnp.zeros((RPT, H), jnp.float32)

    y1, y2, y3 = _sc_spmm3(x0, colp, rowp, valp, zeros)
    e1, e2, e3, total = _assemble(item_emb, y1, y2, y3)
    return (total, (item_emb, e1, e2, e3))
